# baseline (speedup 1.0000x reference)
import functools

import jax
import jax.numpy as jnp
from jax import lax
from jax.experimental import pallas as pl
from jax.experimental.pallas import tpu as pltpu
from jax.experimental.pallas import tpu_sc as plsc

B = 16384
DIM = 16
CITY_ROWS = 1000000
SC_TS = (0, 3)
TINY_TS = (1, 2, 5, 6, 7, 8, 9)
TINY_SIZES = (7, 2, 10, 10, 10, 10, 10)
NSC = len(SC_TS)

_NC = 2
_NS = 16
_NW = _NC * _NS
_RPW = B // _NW


@functools.cache
def _make_sc_a():
    mesh = plsc.VectorSubcoreMesh(core_axis_name="c", subcore_axis_name="s")
    return functools.partial(
        pl.kernel,
        mesh=mesh,
        compiler_params=pltpu.CompilerParams(use_tc_tiling_on_sc=False),
        out_type=jax.ShapeDtypeStruct((B, NSC * DIM), jnp.float32),
        scratch_types=[
            pltpu.VMEM((NSC, _RPW), jnp.int32),
            [pltpu.VMEM((_RPW, DIM), jnp.float32)] * NSC,
            [pltpu.SemaphoreType.DMA] * NSC,
            [pltpu.SemaphoreType.DMA] * NSC,
        ],
    )(_sc_a_body)


def _sc_a_body(t0, t3, idx_hbm, out_hbm, idx_v, bufs, gsems, wsems):
    tables = (t0, t3)
    wid = lax.axis_index("s") * _NC + lax.axis_index("c")
    base = wid * _RPW
    pltpu.sync_copy(idx_hbm.at[:, pl.ds(base, _RPW)], idx_v)
    gcps = [pltpu.async_copy(tables[k].at[idx_v.at[k]], bufs[k], gsems[k])
            for k in range(NSC)]
    wcps = []
    for k in range(NSC):
        gcps[k].wait()
        wcps.append(pltpu.async_copy(
            bufs[k], out_hbm.at[pl.ds(base, _RPW), pl.ds(k * DIM, DIM)],
            wsems[k]))
    for cp in wcps:
        cp.wait()


@functools.cache
def _make_city_gather():
    mesh = plsc.VectorSubcoreMesh(core_axis_name="c", subcore_axis_name="s")
    return functools.partial(
        pl.kernel,
        mesh=mesh,
        compiler_params=pltpu.CompilerParams(use_tc_tiling_on_sc=False),
        out_type=jax.ShapeDtypeStruct((B, DIM), jnp.float32),
        scratch_types=[
            pltpu.VMEM((_RPW,), jnp.int32),
            pltpu.VMEM((_RPW, DIM), jnp.float32),
            pltpu.SemaphoreType.DMA,
        ],
    )(_city_gather_body)


def _city_gather_body(city_hbm, idx_hbm, out_hbm, idx_v, rows_v, sem):
    wid = lax.axis_index("s") * _NC + lax.axis_index("c")
    base = wid * _RPW
    pltpu.sync_copy(idx_hbm.at[pl.ds(base, _RPW)], idx_v)
    pltpu.async_copy(city_hbm.at[idx_v], rows_v, sem).wait()
    pltpu.sync_copy(rows_v, out_hbm.at[pl.ds(base, _RPW), :])


_DCB = 16384


def _detile_body(x_ref, o_ref):
    x2 = x_ref[...].reshape(DIM, _DCB)
    o_ref[...] = x2.T


def _detile_city(city3):
    grid = ((CITY_ROWS + _DCB - 1) // _DCB,)
    return pl.pallas_call(
        _detile_body,
        grid=grid,
        in_specs=[pl.BlockSpec((2, 8, _DCB), lambda i: (0, 0, i))],
        out_specs=pl.BlockSpec((_DCB, DIM), lambda i: (i, 0)),
        out_shape=jax.ShapeDtypeStruct((CITY_ROWS, DIM), jnp.float32),
    )(city3)


def _mlp_body(a_ref, y_ref, ti_ref, l_ref, c_ref,
              tb1, tb2, tb5, tb6, tb7, tb8, tb9,
              w1_ref, b1_ref, w2_ref, b2_ref, o_ref):
    tiny_tbls = (tb1, tb2, tb5, tb6, tb7, tb8, tb9)
    ohs = []
    for k, s in enumerate(TINY_SIZES):
        idx_col = ti_ref[:, k:k + 1]
        iota_row = lax.broadcasted_iota(jnp.int32, (1, s), 1)
        oh = (idx_col == iota_row).astype(jnp.float32)
        ohs.append(jnp.dot(oh, tiny_tbls[k][...],
                           preferred_element_type=jnp.float32))
    x = jnp.concatenate(
        [a_ref[:, :DIM], ohs[0], ohs[1], a_ref[:, DIM:], y_ref[...],
         ohs[2], ohs[3], ohs[4], ohs[5], ohs[6], l_ref[...], c_ref[...]],
        axis=1)
    h = jnp.dot(x, w1_ref[...], preferred_element_type=jnp.float32)
    h = jnp.maximum(h + b1_ref[...], 0.0)
    o = jnp.dot(h, w2_ref[...], preferred_element_type=jnp.float32)
    o_ref[...] = jnp.maximum(o + b2_ref[...], 0.0)


def _mlp(a, city, tiny_idx, logistic, cnn_rnn, tiny_tbls, w1, b1, w2, b2,
         block_m=2048):
    grid = (B // block_m,)
    kin = w1.shape[0]
    return pl.pallas_call(
        _mlp_body,
        grid=grid,
        in_specs=[
            pl.BlockSpec((block_m, NSC * DIM), lambda i: (i, 0)),
            pl.BlockSpec((block_m, DIM), lambda i: (i, 0)),
            pl.BlockSpec((block_m, 8), lambda i: (i, 0)),
            pl.BlockSpec((block_m, 56), lambda i: (i, 0)),
            pl.BlockSpec((block_m, 32), lambda i: (i, 0)),
        ] + [
            pl.BlockSpec((s, DIM), lambda i: (0, 0)) for s in TINY_SIZES
        ] + [
            pl.BlockSpec((kin, 256), lambda i: (0, 0)),
            pl.BlockSpec((1, 256), lambda i: (0, 0)),
            pl.BlockSpec((256, 256), lambda i: (0, 0)),
            pl.BlockSpec((1, 256), lambda i: (0, 0)),
        ],
        out_specs=pl.BlockSpec((block_m, 256), lambda i: (i, 0)),
        out_shape=jax.ShapeDtypeStruct((B, 256), jnp.float32),
    )(a, city, tiny_idx, logistic, cnn_rnn, *tiny_tbls, w1, b1, w2, b2)


def kernel(categ_distance_class, categ_weekday_class, categ_if_busytime_class,
           categ_slice_id_class, categ_city_class, categ_day_before2_type_class,
           categ_day_before1_type_class, categ_day_type_class,
           categ_day_after1_type_class, categ_day_after2_type_class,
           emb_distance_class, emb_weekday_class, emb_if_busytime_class,
           emb_slice_id_class, emb_city_class, emb_day_before2_type_class,
           emb_day_before1_type_class, emb_day_type_class,
           emb_day_after1_type_class, emb_day_after2_type_class,
           logistic, cnn_rnn, W1, b1, W2, b2):
    categs = (categ_distance_class, categ_weekday_class,
              categ_if_busytime_class, categ_slice_id_class, categ_city_class,
              categ_day_before2_type_class, categ_day_before1_type_class,
              categ_day_type_class, categ_day_after1_type_class,
              categ_day_after2_type_class)
    tables = (emb_distance_class, emb_weekday_class, emb_if_busytime_class,
              emb_slice_id_class, emb_city_class, emb_day_before2_type_class,
              emb_day_before1_type_class, emb_day_type_class,
              emb_day_after1_type_class, emb_day_after2_type_class)
    idx_sc = jnp.stack([categs[t].astype(jnp.int32) for t in SC_TS])
    a = _make_sc_a()(*[tables[t] for t in SC_TS], idx_sc)
    city3 = emb_city_class.T.reshape(2, 8, CITY_ROWS)
    city2d = _detile_city(city3)
    city = _make_city_gather()(city2d, categs[4].astype(jnp.int32))
    tiny_idx = jnp.stack(
        [categs[t].astype(jnp.int32) for t in TINY_TS]
        + [jnp.zeros((B,), jnp.int32)], axis=1)
    return _mlp(a, city, tiny_idx, logistic, cnn_rnn,
                [tables[t] for t in TINY_TS],
                W1, b1.reshape(1, 256), W2, b2.reshape(1, 256))


# speedup vs baseline: 2.1535x; 2.1535x over previous
import functools

import jax
import jax.numpy as jnp
from jax import lax
from jax.experimental import pallas as pl
from jax.experimental.pallas import tpu as pltpu
from jax.experimental.pallas import tpu_sc as plsc

B = 16384
DIM = 16
CITY_ROWS = 1000000
SC_TS = (0, 3)
TINY_TS = (1, 2, 5, 6, 7, 8, 9)
TINY_SIZES = (7, 2, 10, 10, 10, 10, 10)
NSC = len(SC_TS)

_NC = 2
_NS = 16
_NW = _NC * _NS
_RPW = B // _NW


@functools.cache
def _make_sc_a():
    mesh = plsc.VectorSubcoreMesh(core_axis_name="c", subcore_axis_name="s")
    return functools.partial(
        pl.kernel,
        mesh=mesh,
        compiler_params=pltpu.CompilerParams(use_tc_tiling_on_sc=False),
        out_type=jax.ShapeDtypeStruct((B, NSC * DIM), jnp.float32),
        scratch_types=[
            pltpu.VMEM((NSC, _RPW), jnp.int32),
            [pltpu.VMEM((_RPW, DIM), jnp.float32)] * NSC,
            [pltpu.SemaphoreType.DMA] * NSC,
            [pltpu.SemaphoreType.DMA] * NSC,
        ],
    )(_sc_a_body)


def _sc_a_body(t0, t3, idx_hbm, out_hbm, idx_v, bufs, gsems, wsems):
    tables = (t0, t3)
    wid = lax.axis_index("s") * _NC + lax.axis_index("c")
    base = wid * _RPW
    pltpu.sync_copy(idx_hbm.at[:, pl.ds(base, _RPW)], idx_v)
    gcps = [pltpu.async_copy(tables[k].at[idx_v.at[k]], bufs[k], gsems[k])
            for k in range(NSC)]
    wcps = []
    for k in range(NSC):
        gcps[k].wait()
        wcps.append(pltpu.async_copy(
            bufs[k], out_hbm.at[pl.ds(base, _RPW), pl.ds(k * DIM, DIM)],
            wsems[k]))
    for cp in wcps:
        cp.wait()


@functools.cache
def _make_city_gather():
    mesh = plsc.VectorSubcoreMesh(core_axis_name="c", subcore_axis_name="s")
    return functools.partial(
        pl.kernel,
        mesh=mesh,
        compiler_params=pltpu.CompilerParams(use_tc_tiling_on_sc=False),
        out_type=jax.ShapeDtypeStruct((B, 128), jnp.float32),
        scratch_types=[
            pltpu.VMEM((_RPW,), jnp.int32),
            pltpu.VMEM((_RPW, 128), jnp.float32),
            pltpu.SemaphoreType.DMA,
        ],
    )(_city_gather_body)


def _city_gather_body(city_hbm, idx_hbm, out_hbm, idx_v, rows_v, sem):
    wid = lax.axis_index("s") * _NC + lax.axis_index("c")
    base = wid * _RPW
    pltpu.sync_copy(idx_hbm.at[pl.ds(base, _RPW)], idx_v)
    pltpu.async_copy(city_hbm.at[idx_v], rows_v, sem).wait()
    pltpu.sync_copy(rows_v, out_hbm.at[pl.ds(base, _RPW), :])


_DCB = 16384


def _detile_body(x_ref, o_ref):
    x2 = x_ref[...].reshape(DIM, _DCB)
    xt = x2.T
    o_ref[...] = jnp.concatenate(
        [xt, jnp.zeros((_DCB, 128 - DIM), jnp.float32)], axis=1)


def _detile_city(city3):
    grid = ((CITY_ROWS + _DCB - 1) // _DCB,)
    return pl.pallas_call(
        _detile_body,
        grid=grid,
        in_specs=[pl.BlockSpec((2, 8, _DCB), lambda i: (0, 0, i))],
        out_specs=pl.BlockSpec((_DCB, 128), lambda i: (i, 0)),
        out_shape=jax.ShapeDtypeStruct((CITY_ROWS, 128), jnp.float32),
    )(city3)


def _mlp_body(a_ref, y_ref, ti_ref, l_ref, c_ref,
              tb1, tb2, tb5, tb6, tb7, tb8, tb9,
              w1_ref, b1_ref, w2_ref, b2_ref, o_ref):
    tiny_tbls = (tb1, tb2, tb5, tb6, tb7, tb8, tb9)
    ohs = []
    for k, s in enumerate(TINY_SIZES):
        idx_col = ti_ref[:, k:k + 1]
        iota_row = lax.broadcasted_iota(jnp.int32, (1, s), 1)
        oh = (idx_col == iota_row).astype(jnp.float32)
        ohs.append(jnp.dot(oh, tiny_tbls[k][...],
                           preferred_element_type=jnp.float32))
    x = jnp.concatenate(
        [a_ref[:, :DIM], ohs[0], ohs[1], a_ref[:, DIM:], y_ref[:, :DIM],
         ohs[2], ohs[3], ohs[4], ohs[5], ohs[6], l_ref[...], c_ref[...]],
        axis=1)
    h = jnp.dot(x, w1_ref[...], preferred_element_type=jnp.float32)
    h = jnp.maximum(h + b1_ref[...], 0.0)
    o = jnp.dot(h, w2_ref[...], preferred_element_type=jnp.float32)
    o_ref[...] = jnp.maximum(o + b2_ref[...], 0.0)


def _mlp(a, city, tiny_idx, logistic, cnn_rnn, tiny_tbls, w1, b1, w2, b2,
         block_m=2048):
    grid = (B // block_m,)
    kin = w1.shape[0]
    return pl.pallas_call(
        _mlp_body,
        grid=grid,
        in_specs=[
            pl.BlockSpec((block_m, NSC * DIM), lambda i: (i, 0)),
            pl.BlockSpec((block_m, 128), lambda i: (i, 0)),
            pl.BlockSpec((block_m, 8), lambda i: (i, 0)),
            pl.BlockSpec((block_m, 56), lambda i: (i, 0)),
            pl.BlockSpec((block_m, 32), lambda i: (i, 0)),
        ] + [
            pl.BlockSpec((s, DIM), lambda i: (0, 0)) for s in TINY_SIZES
        ] + [
            pl.BlockSpec((kin, 256), lambda i: (0, 0)),
            pl.BlockSpec((1, 256), lambda i: (0, 0)),
            pl.BlockSpec((256, 256), lambda i: (0, 0)),
            pl.BlockSpec((1, 256), lambda i: (0, 0)),
        ],
        out_specs=pl.BlockSpec((block_m, 256), lambda i: (i, 0)),
        out_shape=jax.ShapeDtypeStruct((B, 256), jnp.float32),
    )(a, city, tiny_idx, logistic, cnn_rnn, *tiny_tbls, w1, b1, w2, b2)


def kernel(categ_distance_class, categ_weekday_class, categ_if_busytime_class,
           categ_slice_id_class, categ_city_class, categ_day_before2_type_class,
           categ_day_before1_type_class, categ_day_type_class,
           categ_day_after1_type_class, categ_day_after2_type_class,
           emb_distance_class, emb_weekday_class, emb_if_busytime_class,
           emb_slice_id_class, emb_city_class, emb_day_before2_type_class,
           emb_day_before1_type_class, emb_day_type_class,
           emb_day_after1_type_class, emb_day_after2_type_class,
           logistic, cnn_rnn, W1, b1, W2, b2):
    categs = (categ_distance_class, categ_weekday_class,
              categ_if_busytime_class, categ_slice_id_class, categ_city_class,
              categ_day_before2_type_class, categ_day_before1_type_class,
              categ_day_type_class, categ_day_after1_type_class,
              categ_day_after2_type_class)
    tables = (emb_distance_class, emb_weekday_class, emb_if_busytime_class,
              emb_slice_id_class, emb_city_class, emb_day_before2_type_class,
              emb_day_before1_type_class, emb_day_type_class,
              emb_day_after1_type_class, emb_day_after2_type_class)
    idx_sc = jnp.stack([categs[t].astype(jnp.int32) for t in SC_TS])
    a = _make_sc_a()(*[tables[t] for t in SC_TS], idx_sc)
    city3 = emb_city_class.T.reshape(2, 8, CITY_ROWS)
    city2d = _detile_city(city3)
    city = _make_city_gather()(city2d, categs[4].astype(jnp.int32))
    tiny_idx = jnp.stack(
        [categs[t].astype(jnp.int32) for t in TINY_TS]
        + [jnp.zeros((B,), jnp.int32)], axis=1)
    return _mlp(a, city, tiny_idx, logistic, cnn_rnn,
                [tables[t] for t in TINY_TS],
                W1, b1.reshape(1, 256), W2, b2.reshape(1, 256))
